# baseline (speedup 1.0000x reference)
import functools

import numpy as np

import jax
import jax.numpy as jnp
from jax import lax
from jax.experimental import pallas as pl
from jax.experimental.pallas import tpu as pltpu
from jax.experimental.pallas import tpu_sc as plsc

N = 10000
E = 320000
F = 128
HID = 128
CLS = 64

NC, NS, LANES = 2, 16, 16
NTILES = NC * NS
NPAD = 10240
EPAD = 327680
K = 80
BM = 2000

CB = 32
NG0 = 6
NG1 = 2
NCHUNK0 = NG0 * CB
NCHUNK1 = NG1 * CB
E0 = NS * NCHUNK0 * K
E1 = NS * NCHUNK1 * K


def _natcol(nfull):
    m = np.zeros((nfull * 32,), np.int32)
    for q in range(nfull):
        for p in range(16):
            m[32 * q + 2 * p] = 32 * q + p
            m[32 * q + 2 * p + 1] = 32 * q + 16 + p
    return m


def _make_edge_kernel(dp):
    nfull = (dp - 16) // 32
    dwords = (dp + 16) // 2
    aword = nfull * 16 + 1
    rows_per_tile = NPAD // NS
    mesh = plsc.VectorSubcoreMesh(core_axis_name="c", subcore_axis_name="s",
                                  num_cores=NC, num_subcores=NS)
    himask = -65536

    @functools.partial(
        pl.kernel,
        out_type=jax.ShapeDtypeStruct((NC * NPAD, dp), jnp.float32),
        mesh=mesh,
        scratch_types=[
            pltpu.VMEM((CB, K), jnp.int32),
            pltpu.VMEM((CB, K), jnp.int32),
            pltpu.VMEM((CB, K), jnp.float32),
            pltpu.VMEM((K, dwords), jnp.int32),
            pltpu.VMEM((K, dwords), jnp.int32),
            pltpu.VMEM((K, dp), jnp.float32),
            pltpu.VMEM((K,), jnp.float32),
            pltpu.VMEM_SHARED((NPAD, dp), jnp.float32),
            pltpu.SemaphoreType.DMA,
            pltpu.SemaphoreType.DMA,
            pltpu.SemaphoreType.DMA,
        ],
        compiler_params=pltpu.CompilerParams(needs_layout_passes=False,
                                             use_tc_tiling_on_sc=False),
    )
    def edge_kernel(hpad, adst, srcm, dstm, acc_out,
                    src_v, dst_v, adb_v, rows0_v, rows1_v, scaled_v, w_v,
                    acc_s, gs0, gs1, asem):
        cid = lax.axis_index("c")
        sid = lax.axis_index("s")
        wid = cid * NS + sid
        rows_bufs = (rows0_v, rows1_v)
        gsems = (gs0, gs1)

        zero = jnp.zeros((LANES,), jnp.float32)

        def zrow(r, carry):
            for q in range(dp // LANES):
                scaled_v[r, pl.ds(q * LANES, LANES)] = zero
            return carry

        lax.fori_loop(0, K, zrow, None)
        base = sid * rows_per_tile
        for k in range(rows_per_tile // K):
            pltpu.sync_copy(scaled_v, acc_s.at[pl.ds(base + k * K, K)])
        plsc.subcore_barrier()

        lanes_iota = lax.iota(jnp.int32, LANES)

        def issue(cc, b):
            pltpu.async_copy(hpad.at[src_v.at[cc]], rows_bufs[b], gsems[b])

        def wait(b):
            pltpu.make_async_copy(hpad.at[src_v.at[0]], rows_bufs[b],
                                  gsems[b]).wait()

        def compute(cc, b):
            rows_v = rows_bufs[b]
            for j in range(K // LANES):
                xw = plsc.load_gather(
                    rows_v, [lanes_iota + (j * LANES),
                             jnp.full((LANES,), aword, jnp.int32)])
                asv = plsc.bitcast(xw << 16, jnp.float32)
                e = asv + adb_v[cc, pl.ds(j * LANES, LANES)]
                w = jnp.exp(jnp.maximum(e, 0.2 * e))
                w_v[pl.ds(j * LANES, LANES)] = w

            def scale(r, carry3):
                wr = plsc.load_gather(w_v, [jnp.full((LANES,), r, jnp.int32)])
                for q in range(nfull):
                    x = rows_v[r, pl.ds(q * LANES, LANES)]
                    lo = plsc.bitcast(x << 16, jnp.float32)
                    hi = plsc.bitcast(x & himask, jnp.float32)
                    scaled_v[r, pl.ds(32 * q, LANES)] = lo * wr
                    scaled_v[r, pl.ds(32 * q + 16, LANES)] = hi * wr
                xt = rows_v[r, pl.ds(nfull * LANES, LANES)]
                lot = plsc.bitcast(xt << 16, jnp.float32)
                scaled_v[r, pl.ds(32 * nfull, LANES)] = lot * wr
                return carry3

            lax.fori_loop(0, K, scale, None)
            pltpu.sync_copy(scaled_v, acc_s.at[dst_v.at[cc]], add=True)

        def group(gb, carry):
            pltpu.sync_copy(srcm.at[wid, pl.ds(gb * CB, CB)], src_v)
            pltpu.sync_copy(dstm.at[wid, pl.ds(gb * CB, CB)], dst_v)
            for i in range(CB):
                pltpu.async_copy(adst.at[dst_v.at[i]], adb_v.at[i], asem)
            issue(0, 0)
            for i in range(CB):
                pltpu.make_async_copy(adst.at[dst_v.at[0]], adb_v.at[0],
                                      asem).wait()

            def pair(g, carry2):
                c0 = 2 * g
                issue(c0 + 1, 1)
                wait(0)
                compute(c0, 0)

                @pl.when(g < CB // 2 - 1)
                def _():
                    issue(c0 + 2, 0)

                wait(1)
                compute(c0 + 1, 1)
                return carry2

            lax.fori_loop(0, CB // 2, pair, None)
            return carry

        ngroup = jnp.where(cid == 0, NG0, NG1)
        lax.fori_loop(0, ngroup, group, None)
        plsc.subcore_barrier()
        pltpu.sync_copy(acc_s.at[pl.ds(base, rows_per_tile)],
                        acc_out.at[pl.ds(cid * NPAD + base, rows_per_tile)])

    return edge_kernel


_edge_l1 = _make_edge_kernel(HID + 16)
_edge_l2 = _make_edge_kernel(CLS + 16)


def _dense1_body(x_ref, w_ref, wp_ref, asv_ref, adv_ref,
                 hq_ref, alph_ref, hn_ref):
    x = x_ref[...]
    hn = jnp.dot(x, w_ref[...], preferred_element_type=jnp.float32)
    hq = jnp.dot(x, wp_ref[...], preferred_element_type=jnp.float32)
    asrc = jnp.sum(hn * asv_ref[...], axis=1, keepdims=True)
    adst = jnp.sum(hn * adv_ref[...], axis=1, keepdims=True)
    e = asrc + adst
    wself = jnp.exp(jnp.maximum(e, 0.2 * e))
    bm = hn.shape[0]
    hq_ref[...] = jnp.concatenate(
        [hq, jnp.ones((bm, 1), jnp.float32), jnp.zeros((bm, 1), jnp.float32),
         asrc, jnp.zeros((bm, 29), jnp.float32)], axis=1).astype(jnp.bfloat16)
    alph_ref[...] = jnp.concatenate(
        [asrc, adst, wself, jnp.zeros((bm, 5), jnp.float32)], axis=1)
    hn_ref[...] = hn


def _dense1(x, W1, W1p, asv, adv):
    return pl.pallas_call(
        _dense1_body,
        grid=(N // BM,),
        in_specs=[
            pl.BlockSpec((BM, F), lambda i: (i, 0)),
            pl.BlockSpec((F, HID), lambda i: (0, 0)),
            pl.BlockSpec((F, HID), lambda i: (0, 0)),
            pl.BlockSpec((1, HID), lambda i: (0, 0)),
            pl.BlockSpec((1, HID), lambda i: (0, 0)),
        ],
        out_specs=[
            pl.BlockSpec((BM, HID + 32), lambda i: (i, 0)),
            pl.BlockSpec((BM, 8), lambda i: (i, 0)),
            pl.BlockSpec((BM, HID), lambda i: (i, 0)),
        ],
        out_shape=[
            jax.ShapeDtypeStruct((N, HID + 32), jnp.bfloat16),
            jax.ShapeDtypeStruct((N, 8), jnp.float32),
            jax.ShapeDtypeStruct((N, HID), jnp.float32),
        ],
    )(x, W1, W1p, asv, adv)


def _mid_body(a0_ref, a1_ref, alph_ref, hn_ref, b1_ref, w2_ref, w2p_ref,
              asv_ref, adv_ref, hq2_ref, alph2_ref, h2n_ref):
    wself = alph_ref[:, 2:3]
    num = a0_ref[:, :HID] + a1_ref[:, :HID] + wself * hn_ref[...]
    den = (a0_ref[:, HID:HID + 1] + a1_ref[:, HID:HID + 1] + wself + 1e-16)
    z = jnp.maximum(num / den + b1_ref[...], 0.0)
    h2n = jnp.dot(z, w2_ref[...], preferred_element_type=jnp.float32)
    h2q = jnp.dot(z, w2p_ref[...], preferred_element_type=jnp.float32)
    asrc2 = jnp.sum(h2n * asv_ref[...], axis=1, keepdims=True)
    adst2 = jnp.sum(h2n * adv_ref[...], axis=1, keepdims=True)
    e2 = asrc2 + adst2
    wself2 = jnp.exp(jnp.maximum(e2, 0.2 * e2))
    bm = h2n.shape[0]
    hq2_ref[...] = jnp.concatenate(
        [h2q, jnp.ones((bm, 1), jnp.float32), jnp.zeros((bm, 1), jnp.float32),
         asrc2, jnp.zeros((bm, 29), jnp.float32)], axis=1).astype(jnp.bfloat16)
    alph2_ref[...] = jnp.concatenate(
        [asrc2, adst2, wself2, jnp.zeros((bm, 5), jnp.float32)], axis=1)
    h2n_ref[...] = h2n


def _mid(a0, a1, alph, hn, b1, W2, W2p, asv2, adv2):
    return pl.pallas_call(
        _mid_body,
        grid=(N // BM,),
        in_specs=[
            pl.BlockSpec((BM, HID + 16), lambda i: (i, 0)),
            pl.BlockSpec((BM, HID + 16), lambda i: (i, 0)),
            pl.BlockSpec((BM, 8), lambda i: (i, 0)),
            pl.BlockSpec((BM, HID), lambda i: (i, 0)),
            pl.BlockSpec((1, HID), lambda i: (0, 0)),
            pl.BlockSpec((HID, CLS), lambda i: (0, 0)),
            pl.BlockSpec((HID, CLS), lambda i: (0, 0)),
            pl.BlockSpec((1, CLS), lambda i: (0, 0)),
            pl.BlockSpec((1, CLS), lambda i: (0, 0)),
        ],
        out_specs=[
            pl.BlockSpec((BM, CLS + 32), lambda i: (i, 0)),
            pl.BlockSpec((BM, 8), lambda i: (i, 0)),
            pl.BlockSpec((BM, CLS), lambda i: (i, 0)),
        ],
        out_shape=[
            jax.ShapeDtypeStruct((N, CLS + 32), jnp.bfloat16),
            jax.ShapeDtypeStruct((N, 8), jnp.float32),
            jax.ShapeDtypeStruct((N, CLS), jnp.float32),
        ],
    )(a0, a1, alph, hn, b1, W2, W2p, asv2, adv2)


def _final_body(a0_ref, a1_ref, alph2_ref, h2n_ref, b2_ref, out_ref):
    wself = alph2_ref[:, 2:3]
    num = a0_ref[:, :CLS] + a1_ref[:, :CLS] + wself * h2n_ref[...]
    den = (a0_ref[:, CLS:CLS + 1] + a1_ref[:, CLS:CLS + 1] + wself + 1e-16)
    o = num / den + b2_ref[...]
    m = jnp.max(o, axis=1, keepdims=True)
    s = o - m
    out_ref[...] = s - jnp.log(jnp.sum(jnp.exp(s), axis=1, keepdims=True))


def _final(a0, a1, alph2, h2n, b2):
    return pl.pallas_call(
        _final_body,
        grid=(N // BM,),
        in_specs=[
            pl.BlockSpec((BM, CLS + 16), lambda i: (i, 0)),
            pl.BlockSpec((BM, CLS + 16), lambda i: (i, 0)),
            pl.BlockSpec((BM, 8), lambda i: (i, 0)),
            pl.BlockSpec((BM, CLS), lambda i: (i, 0)),
            pl.BlockSpec((1, CLS), lambda i: (0, 0)),
        ],
        out_specs=pl.BlockSpec((BM, CLS), lambda i: (i, 0)),
        out_shape=jax.ShapeDtypeStruct((N, CLS), jnp.float32),
    )(a0, a1, alph2, h2n, b2)


def kernel(x, edge_index, W1, a_src1, a_dst1, b1, W2, a_src2, a_dst2, b2):
    src = edge_index[0]
    dst = edge_index[1]
    pad_e = EPAD - E
    src_p = jnp.concatenate([src, jnp.zeros((pad_e,), jnp.int32)])
    dst_p = jnp.concatenate([dst, jnp.full((pad_e,), N, jnp.int32)])

    def _split(a):
        a0 = a[:E0].reshape(NS, NCHUNK0, K)
        a1 = jnp.pad(a[E0:].reshape(NS, NCHUNK1, K),
                     ((0, 0), (0, NCHUNK0 - NCHUNK1), (0, 0)))
        return jnp.concatenate([a0, a1], axis=0)

    srcm = _split(src_p)
    dstm = _split(dst_p)

    W1p = W1[:, _natcol(HID // 32)]
    W2p = W2[:, _natcol(CLS // 32)]

    hq1, alph1, hn1 = _dense1(x, W1, W1p, a_src1, a_dst1)
    hq1_i32 = lax.bitcast_convert_type(
        hq1.reshape(N, (HID + 32) // 2, 2), jnp.int32)
    adst1t = jnp.pad(alph1[:, 1], (0, NPAD - N))
    acc1 = _edge_l1(hq1_i32, adst1t, srcm, dstm)

    hq2, alph2, h2n = _mid(acc1[:NPAD], acc1[NPAD:], alph1, hn1,
                           b1.reshape(1, HID), W2, W2p, a_src2, a_dst2)
    hq2_i32 = lax.bitcast_convert_type(
        hq2.reshape(N, (CLS + 32) // 2, 2), jnp.int32)
    adst2t = jnp.pad(alph2[:, 1], (0, NPAD - N))
    acc2 = _edge_l2(hq2_i32, adst2t, srcm, dstm)

    return _final(acc2[:NPAD], acc2[NPAD:], alph2, h2n, b2.reshape(1, CLS))


# speedup vs baseline: 1.6434x; 1.6434x over previous
import functools

import numpy as np

import jax
import jax.numpy as jnp
from jax import lax
from jax.experimental import pallas as pl
from jax.experimental.pallas import tpu as pltpu
from jax.experimental.pallas import tpu_sc as plsc

N = 10000
E = 320000
F = 128
HID = 128
CLS = 64

NC, NS, LANES = 2, 16, 16
NTILES = NC * NS
NPAD = 10240
EPAD = 327680
K = 80
BM = 2000

CB = 32
NG0 = 6
NG1 = 2
NCHUNK0 = NG0 * CB
NCHUNK1 = NG1 * CB
E0 = NS * NCHUNK0 * K
E1 = NS * NCHUNK1 * K


def _natcol(nfull):
    m = np.zeros((nfull * 32,), np.int32)
    for q in range(nfull):
        for p in range(16):
            m[32 * q + 2 * p] = 32 * q + p
            m[32 * q + 2 * p + 1] = 32 * q + 16 + p
    return m


def _make_edge_kernel(dp):
    nfull = (dp - 16) // 32
    dwords = (dp + 16) // 2
    aword = nfull * 16 + 1
    rows_per_tile = NPAD // NS
    mesh = plsc.VectorSubcoreMesh(core_axis_name="c", subcore_axis_name="s",
                                  num_cores=NC, num_subcores=NS)
    himask = -65536

    @functools.partial(
        pl.kernel,
        out_type=jax.ShapeDtypeStruct((NC * NPAD, dp), jnp.float32),
        mesh=mesh,
        scratch_types=[
            pltpu.VMEM((CB, K), jnp.int32),
            pltpu.VMEM((CB, K), jnp.int32),
            pltpu.VMEM((CB, K), jnp.float32),
            pltpu.VMEM((K, dwords), jnp.int32),
            pltpu.VMEM((K, dwords), jnp.int32),
            pltpu.VMEM((K, dp), jnp.float32),
            pltpu.VMEM((K,), jnp.float32),
            pltpu.VMEM_SHARED((NPAD, dp), jnp.float32),
            pltpu.SemaphoreType.DMA,
            pltpu.SemaphoreType.DMA,
            pltpu.SemaphoreType.DMA,
        ],
        compiler_params=pltpu.CompilerParams(needs_layout_passes=False,
                                             use_tc_tiling_on_sc=False),
    )
    def edge_kernel(hpad, adst, srcm, dstm, acc_out,
                    src_v, dst_v, adb_v, rows0_v, rows1_v, scaled_v, w_v,
                    acc_s, gs0, gs1, asem):
        cid = lax.axis_index("c")
        sid = lax.axis_index("s")
        wid = cid * NS + sid
        rows_bufs = (rows0_v, rows1_v)
        gsems = (gs0, gs1)

        zero = jnp.zeros((LANES,), jnp.float32)

        def zrow(r, carry):
            for q in range(dp // LANES):
                scaled_v[r, pl.ds(q * LANES, LANES)] = zero
            return carry

        lax.fori_loop(0, K, zrow, None)
        base = sid * rows_per_tile
        for k in range(rows_per_tile // K):
            pltpu.sync_copy(scaled_v, acc_s.at[pl.ds(base + k * K, K)])
        plsc.subcore_barrier()

        lanes_iota = lax.iota(jnp.int32, LANES)

        def issue(cc, b):
            pltpu.async_copy(hpad.at[src_v.at[cc]], rows_bufs[b], gsems[b])

        def wait(b):
            pltpu.make_async_copy(hpad.at[src_v.at[0]], rows_bufs[b],
                                  gsems[b]).wait()

        def compute(cc, b):
            rows_v = rows_bufs[b]
            for j in range(K // LANES):
                xw = plsc.load_gather(
                    rows_v, [lanes_iota + (j * LANES),
                             jnp.full((LANES,), aword, jnp.int32)])
                asv = plsc.bitcast(xw << 16, jnp.float32)
                e = asv + adb_v[cc, pl.ds(j * LANES, LANES)]
                w = jnp.exp(jnp.maximum(e, 0.2 * e))
                w_v[pl.ds(j * LANES, LANES)] = w

            @plsc.parallel_loop(0, K, 1, unroll=2)
            def _scale(r):
                wr = plsc.load_gather(w_v, [jnp.full((LANES,), r, jnp.int32)])
                xs = [rows_v[r, pl.ds(q * LANES, LANES)]
                      for q in range(nfull + 1)]
                for q in range(nfull):
                    lo = plsc.bitcast(xs[q] << 16, jnp.float32)
                    hi = plsc.bitcast(xs[q] & himask, jnp.float32)
                    scaled_v[r, pl.ds(32 * q, LANES)] = lo * wr
                    scaled_v[r, pl.ds(32 * q + 16, LANES)] = hi * wr
                lot = plsc.bitcast(xs[nfull] << 16, jnp.float32)
                scaled_v[r, pl.ds(32 * nfull, LANES)] = lot * wr
            pltpu.sync_copy(scaled_v, acc_s.at[dst_v.at[cc]], add=True)

        def group(gb, carry):
            pltpu.sync_copy(srcm.at[wid, pl.ds(gb * CB, CB)], src_v)
            pltpu.sync_copy(dstm.at[wid, pl.ds(gb * CB, CB)], dst_v)
            for i in range(CB):
                pltpu.async_copy(adst.at[dst_v.at[i]], adb_v.at[i], asem)
            issue(0, 0)
            for i in range(CB):
                pltpu.make_async_copy(adst.at[dst_v.at[0]], adb_v.at[0],
                                      asem).wait()

            def pair(g, carry2):
                c0 = 2 * g
                issue(c0 + 1, 1)
                wait(0)
                compute(c0, 0)

                @pl.when(g < CB // 2 - 1)
                def _():
                    issue(c0 + 2, 0)

                wait(1)
                compute(c0 + 1, 1)
                return carry2

            lax.fori_loop(0, CB // 2, pair, None)
            return carry

        ngroup = jnp.where(cid == 0, NG0, NG1)
        lax.fori_loop(0, ngroup, group, None)
        plsc.subcore_barrier()
        pltpu.sync_copy(acc_s.at[pl.ds(base, rows_per_tile)],
                        acc_out.at[pl.ds(cid * NPAD + base, rows_per_tile)])

    return edge_kernel


_edge_l1 = _make_edge_kernel(HID + 16)
_edge_l2 = _make_edge_kernel(CLS + 16)


def _dense1_body(x_ref, w_ref, wp_ref, asv_ref, adv_ref,
                 hq_ref, alph_ref, hn_ref):
    x = x_ref[...]
    hn = jnp.dot(x, w_ref[...], preferred_element_type=jnp.float32)
    hq = jnp.dot(x, wp_ref[...], preferred_element_type=jnp.float32)
    asrc = jnp.sum(hn * asv_ref[...], axis=1, keepdims=True)
    adst = jnp.sum(hn * adv_ref[...], axis=1, keepdims=True)
    e = asrc + adst
    wself = jnp.exp(jnp.maximum(e, 0.2 * e))
    bm = hn.shape[0]
    hq_ref[...] = jnp.concatenate(
        [hq, jnp.ones((bm, 1), jnp.float32), jnp.zeros((bm, 1), jnp.float32),
         asrc, jnp.zeros((bm, 29), jnp.float32)], axis=1).astype(jnp.bfloat16)
    alph_ref[...] = jnp.concatenate(
        [asrc, adst, wself, jnp.zeros((bm, 5), jnp.float32)], axis=1)
    hn_ref[...] = hn


def _dense1(x, W1, W1p, asv, adv):
    return pl.pallas_call(
        _dense1_body,
        grid=(N // BM,),
        in_specs=[
            pl.BlockSpec((BM, F), lambda i: (i, 0)),
            pl.BlockSpec((F, HID), lambda i: (0, 0)),
            pl.BlockSpec((F, HID), lambda i: (0, 0)),
            pl.BlockSpec((1, HID), lambda i: (0, 0)),
            pl.BlockSpec((1, HID), lambda i: (0, 0)),
        ],
        out_specs=[
            pl.BlockSpec((BM, HID + 32), lambda i: (i, 0)),
            pl.BlockSpec((BM, 8), lambda i: (i, 0)),
            pl.BlockSpec((BM, HID), lambda i: (i, 0)),
        ],
        out_shape=[
            jax.ShapeDtypeStruct((N, HID + 32), jnp.bfloat16),
            jax.ShapeDtypeStruct((N, 8), jnp.float32),
            jax.ShapeDtypeStruct((N, HID), jnp.float32),
        ],
    )(x, W1, W1p, asv, adv)


def _mid_body(a0_ref, a1_ref, alph_ref, hn_ref, b1_ref, w2_ref, w2p_ref,
              asv_ref, adv_ref, hq2_ref, alph2_ref, h2n_ref):
    wself = alph_ref[:, 2:3]
    num = a0_ref[:, :HID] + a1_ref[:, :HID] + wself * hn_ref[...]
    den = (a0_ref[:, HID:HID + 1] + a1_ref[:, HID:HID + 1] + wself + 1e-16)
    z = jnp.maximum(num / den + b1_ref[...], 0.0)
    h2n = jnp.dot(z, w2_ref[...], preferred_element_type=jnp.float32)
    h2q = jnp.dot(z, w2p_ref[...], preferred_element_type=jnp.float32)
    asrc2 = jnp.sum(h2n * asv_ref[...], axis=1, keepdims=True)
    adst2 = jnp.sum(h2n * adv_ref[...], axis=1, keepdims=True)
    e2 = asrc2 + adst2
    wself2 = jnp.exp(jnp.maximum(e2, 0.2 * e2))
    bm = h2n.shape[0]
    hq2_ref[...] = jnp.concatenate(
        [h2q, jnp.ones((bm, 1), jnp.float32), jnp.zeros((bm, 1), jnp.float32),
         asrc2, jnp.zeros((bm, 29), jnp.float32)], axis=1).astype(jnp.bfloat16)
    alph2_ref[...] = jnp.concatenate(
        [asrc2, adst2, wself2, jnp.zeros((bm, 5), jnp.float32)], axis=1)
    h2n_ref[...] = h2n


def _mid(a0, a1, alph, hn, b1, W2, W2p, asv2, adv2):
    return pl.pallas_call(
        _mid_body,
        grid=(N // BM,),
        in_specs=[
            pl.BlockSpec((BM, HID + 16), lambda i: (i, 0)),
            pl.BlockSpec((BM, HID + 16), lambda i: (i, 0)),
            pl.BlockSpec((BM, 8), lambda i: (i, 0)),
            pl.BlockSpec((BM, HID), lambda i: (i, 0)),
            pl.BlockSpec((1, HID), lambda i: (0, 0)),
            pl.BlockSpec((HID, CLS), lambda i: (0, 0)),
            pl.BlockSpec((HID, CLS), lambda i: (0, 0)),
            pl.BlockSpec((1, CLS), lambda i: (0, 0)),
            pl.BlockSpec((1, CLS), lambda i: (0, 0)),
        ],
        out_specs=[
            pl.BlockSpec((BM, CLS + 32), lambda i: (i, 0)),
            pl.BlockSpec((BM, 8), lambda i: (i, 0)),
            pl.BlockSpec((BM, CLS), lambda i: (i, 0)),
        ],
        out_shape=[
            jax.ShapeDtypeStruct((N, CLS + 32), jnp.bfloat16),
            jax.ShapeDtypeStruct((N, 8), jnp.float32),
            jax.ShapeDtypeStruct((N, CLS), jnp.float32),
        ],
    )(a0, a1, alph, hn, b1, W2, W2p, asv2, adv2)


def _final_body(a0_ref, a1_ref, alph2_ref, h2n_ref, b2_ref, out_ref):
    wself = alph2_ref[:, 2:3]
    num = a0_ref[:, :CLS] + a1_ref[:, :CLS] + wself * h2n_ref[...]
    den = (a0_ref[:, CLS:CLS + 1] + a1_ref[:, CLS:CLS + 1] + wself + 1e-16)
    o = num / den + b2_ref[...]
    m = jnp.max(o, axis=1, keepdims=True)
    s = o - m
    out_ref[...] = s - jnp.log(jnp.sum(jnp.exp(s), axis=1, keepdims=True))


def _final(a0, a1, alph2, h2n, b2):
    return pl.pallas_call(
        _final_body,
        grid=(N // BM,),
        in_specs=[
            pl.BlockSpec((BM, CLS + 16), lambda i: (i, 0)),
            pl.BlockSpec((BM, CLS + 16), lambda i: (i, 0)),
            pl.BlockSpec((BM, 8), lambda i: (i, 0)),
            pl.BlockSpec((BM, CLS), lambda i: (i, 0)),
            pl.BlockSpec((1, CLS), lambda i: (0, 0)),
        ],
        out_specs=pl.BlockSpec((BM, CLS), lambda i: (i, 0)),
        out_shape=jax.ShapeDtypeStruct((N, CLS), jnp.float32),
    )(a0, a1, alph2, h2n, b2)


def kernel(x, edge_index, W1, a_src1, a_dst1, b1, W2, a_src2, a_dst2, b2):
    src = edge_index[0]
    dst = edge_index[1]
    pad_e = EPAD - E
    src_p = jnp.concatenate([src, jnp.zeros((pad_e,), jnp.int32)])
    dst_p = jnp.concatenate([dst, jnp.full((pad_e,), N, jnp.int32)])

    def _split(a):
        a0 = a[:E0].reshape(NS, NCHUNK0, K)
        a1 = jnp.pad(a[E0:].reshape(NS, NCHUNK1, K),
                     ((0, 0), (0, NCHUNK0 - NCHUNK1), (0, 0)))
        return jnp.concatenate([a0, a1], axis=0)

    srcm = _split(src_p)
    dstm = _split(dst_p)

    W1p = W1[:, _natcol(HID // 32)]
    W2p = W2[:, _natcol(CLS // 32)]

    hq1, alph1, hn1 = _dense1(x, W1, W1p, a_src1, a_dst1)
    hq1_i32 = lax.bitcast_convert_type(
        hq1.reshape(N, (HID + 32) // 2, 2), jnp.int32)
    adst1t = jnp.pad(alph1[:, 1], (0, NPAD - N))
    acc1 = _edge_l1(hq1_i32, adst1t, srcm, dstm)

    hq2, alph2, h2n = _mid(acc1[:NPAD], acc1[NPAD:], alph1, hn1,
                           b1.reshape(1, HID), W2, W2p, a_src2, a_dst2)
    hq2_i32 = lax.bitcast_convert_type(
        hq2.reshape(N, (CLS + 32) // 2, 2), jnp.int32)
    adst2t = jnp.pad(alph2[:, 1], (0, NPAD - N))
    acc2 = _edge_l2(hq2_i32, adst2t, srcm, dstm)

    return _final(acc2[:NPAD], acc2[NPAD:], alph2, h2n, b2.reshape(1, CLS))
